# baseline (speedup 1.0000x reference)
import functools

import jax
import jax.numpy as jnp
from jax import lax
from jax.experimental import pallas as pl
from jax.experimental.pallas import tpu as pltpu
from jax.experimental.pallas import tpu_sc as plsc

_SC_CORES = 2
_SC_SUBCORES = 16
_SC_WORKERS = _SC_CORES * _SC_SUBCORES

_BLK = 64


def _make_sc_row_gather(T_rows, D, R):
    per_w = T_rows // _SC_WORKERS
    n_chunks = per_w // R
    mesh = plsc.VectorSubcoreMesh(core_axis_name="c", subcore_axis_name="s")

    @functools.partial(
        pl.kernel,
        out_type=jax.ShapeDtypeStruct((T_rows, D), jnp.float32),
        mesh=mesh,
        scratch_types=[
            pltpu.VMEM((R,), jnp.int32),
            pltpu.VMEM((R, D), jnp.float32),
            pltpu.SemaphoreType.DMA,
        ],
    )
    def gather_kernel(src_hbm, idx_hbm, out_hbm, idx_v, rows_v, sem):
        wid = lax.axis_index("s") * _SC_CORES + lax.axis_index("c")
        for c in range(n_chunks):
            base = wid * per_w + c * R
            pltpu.sync_copy(idx_hbm.at[pl.ds(base, R)], idx_v)
            pltpu.async_copy(src_hbm.at[idx_v], rows_v, sem).wait()
            pltpu.sync_copy(rows_v, out_hbm.at[pl.ds(base, R)])

    return gather_kernel


def _ragged_matmul_body(off_r, ord_r, wsc_r, xs_ref, w_ref, o_ref):
    e = pl.program_id(0)
    start = off_r[e]
    end = off_r[e + 1]
    blk0 = start // _BLK
    n_chunks = (end + _BLK - 1) // _BLK - blk0

    def chunk(j, carry):
        s0 = (blk0 + j) * _BLK
        y = jnp.dot(
            xs_ref[pl.ds(s0, _BLK), :], w_ref[0],
            preferred_element_type=jnp.float32,
        )
        for i in range(_BLK):
            g = s0 + i

            @pl.when((g >= start) & (g < end))
            def _():
                t = ord_r[g]
                o_ref[pl.ds(t, 1), :] = y[i : i + 1, :] * wsc_r[t]

        return carry

    lax.fori_loop(0, n_chunks, chunk, 0)


def _ragged_matmul(xs, W, offsets_ext, order, wvec):
    T, K = xs.shape
    E, _, H = W.shape
    grid_spec = pltpu.PrefetchScalarGridSpec(
        num_scalar_prefetch=3,
        grid=(E,),
        in_specs=[
            pl.BlockSpec((T, K), lambda e, off, ordr, wsc: (0, 0)),
            pl.BlockSpec((1, K, H), lambda e, off, ordr, wsc: (e, 0, 0)),
        ],
        out_specs=pl.BlockSpec((T, H), lambda e, off, ordr, wsc: (0, 0)),
    )
    return pl.pallas_call(
        _ragged_matmul_body,
        grid_spec=grid_spec,
        out_shape=jax.ShapeDtypeStruct((T, H), jnp.float32),
    )(offsets_ext, order, wvec, xs, W)


def kernel(intermediate_states, down_weight, full_topk_ids, full_topk_weight):
    x = intermediate_states
    W = down_weight
    T, K = x.shape
    E, _, H = W.shape

    flat_ids = full_topk_ids.reshape(T).astype(jnp.int32)
    order = jnp.argsort(flat_ids).astype(jnp.int32)
    cmp = flat_ids[None, :] < jnp.arange(1, E + 1, dtype=jnp.int32)[:, None]
    offsets_ext = jnp.concatenate(
        [jnp.zeros((1,), jnp.int32), cmp.sum(axis=1).astype(jnp.int32)]
    )
    wvec = full_topk_weight.astype(jnp.float32).reshape(T)
    xs = _make_sc_row_gather(T, K, 64)(x, order)
    out = _ragged_matmul(xs, W, offsets_ext, order, wvec)
    return out


# speedup vs baseline: 1.0405x; 1.0405x over previous
import functools

import jax
import jax.numpy as jnp
from jax import lax
from jax.experimental import pallas as pl
from jax.experimental.pallas import tpu as pltpu
from jax.experimental.pallas import tpu_sc as plsc

_SC_CORES = 2
_SC_SUBCORES = 16
_SC_WORKERS = _SC_CORES * _SC_SUBCORES

_BLK = 32


def _make_sc_row_gather(T_rows, D, R):
    per_w = T_rows // _SC_WORKERS
    n_chunks = per_w // R
    mesh = plsc.VectorSubcoreMesh(core_axis_name="c", subcore_axis_name="s")

    @functools.partial(
        pl.kernel,
        out_type=jax.ShapeDtypeStruct((T_rows, D), jnp.float32),
        mesh=mesh,
        scratch_types=[
            pltpu.VMEM((n_chunks, R), jnp.int32),
            pltpu.VMEM((R, D), jnp.float32),
            pltpu.VMEM((R, D), jnp.float32),
            pltpu.SemaphoreType.DMA,
            pltpu.SemaphoreType.DMA,
        ],
    )
    def gather_kernel(src_hbm, idx_hbm, out_hbm, idx_v, rows_a, rows_b, sem_a, sem_b):
        wid = lax.axis_index("s") * _SC_CORES + lax.axis_index("c")
        pltpu.sync_copy(idx_hbm.at[pl.ds(wid * n_chunks, n_chunks)], idx_v)
        bufs = (rows_a, rows_b)
        sems = (sem_a, sem_b)
        cps = [None, None]
        for c in range(min(2, n_chunks)):
            cps[c % 2] = pltpu.async_copy(
                src_hbm.at[idx_v.at[c]], bufs[c % 2], sems[c % 2]
            )
        for c in range(n_chunks):
            cps[c % 2].wait()
            pltpu.sync_copy(bufs[c % 2], out_hbm.at[pl.ds(wid * per_w + c * R, R)])
            nxt = c + 2
            if nxt < n_chunks:
                cps[nxt % 2] = pltpu.async_copy(
                    src_hbm.at[idx_v.at[nxt]], bufs[nxt % 2], sems[nxt % 2]
                )

    return gather_kernel


def _make_sc_row_and_scalar_gather(T_rows, D, R):
    per_w = T_rows // _SC_WORKERS
    n_chunks = per_w // R
    lanes = 128
    mesh = plsc.VectorSubcoreMesh(core_axis_name="c", subcore_axis_name="s")

    @functools.partial(
        pl.kernel,
        out_type=(
            jax.ShapeDtypeStruct((T_rows, D), jnp.float32),
            jax.ShapeDtypeStruct((T_rows, lanes), jnp.float32),
        ),
        mesh=mesh,
        scratch_types=[
            pltpu.VMEM((R,), jnp.int32),
            pltpu.VMEM((R, D), jnp.float32),
            pltpu.VMEM((R, lanes), jnp.float32),
            pltpu.SemaphoreType.DMA,
            pltpu.SemaphoreType.DMA,
        ],
    )
    def gather_kernel(src_hbm, idx_hbm, svec_hbm, rows_out, s_out,
                      idx_v, rows_v, sg_v, sem, sem2):
        wid = lax.axis_index("s") * _SC_CORES + lax.axis_index("c")
        for c in range(n_chunks):
            base = wid * per_w + c * R
            pltpu.sync_copy(idx_hbm.at[pl.ds(base, R)], idx_v)
            cp = pltpu.async_copy(src_hbm.at[idx_v], rows_v, sem)
            cp2 = pltpu.async_copy(svec_hbm.at[idx_v], sg_v, sem2)
            cp.wait()
            cp2.wait()
            pltpu.sync_copy(rows_v, rows_out.at[pl.ds(base, R)])
            pltpu.sync_copy(sg_v, s_out.at[pl.ds(base, R)])

    return gather_kernel


def _ragged_matmul_body(off_r, xs_ref, w_ref, ws_ref, o_ref):
    e = pl.program_id(0)
    start = off_r[e]
    end = off_r[e + 1]
    blk0 = start // _BLK
    n_chunks = (end + _BLK - 1) // _BLK - blk0

    def chunk(j, carry):
        s0 = (blk0 + j) * _BLK
        y = jnp.dot(
            xs_ref[pl.ds(s0, _BLK), :], w_ref[0],
            preferred_element_type=jnp.float32,
        )
        y = y * ws_ref[pl.ds(s0, _BLK), :]
        r = s0 + lax.broadcasted_iota(jnp.int32, (_BLK, 1), 0)
        mask = (r >= start) & (r < end)
        o_ref[pl.ds(s0, _BLK), :] = jnp.where(mask, y, o_ref[pl.ds(s0, _BLK), :])
        return carry

    lax.fori_loop(0, n_chunks, chunk, 0)


def _ragged_matmul(xs, W, ws, offsets_ext):
    T, K = xs.shape
    E, _, H = W.shape
    grid_spec = pltpu.PrefetchScalarGridSpec(
        num_scalar_prefetch=1,
        grid=(E,),
        in_specs=[
            pl.BlockSpec((T, K), lambda e, off: (0, 0)),
            pl.BlockSpec((1, K, H), lambda e, off: (e, 0, 0)),
            pl.BlockSpec((T, 1), lambda e, off: (0, 0)),
        ],
        out_specs=pl.BlockSpec((T, H), lambda e, off: (0, 0)),
    )
    return pl.pallas_call(
        _ragged_matmul_body,
        grid_spec=grid_spec,
        out_shape=jax.ShapeDtypeStruct((T, H), jnp.float32),
    )(offsets_ext, xs, W, ws)


def kernel(intermediate_states, down_weight, full_topk_ids, full_topk_weight):
    x = intermediate_states
    W = down_weight
    T, K = x.shape
    E, _, H = W.shape

    flat_ids = full_topk_ids.reshape(T).astype(jnp.int32)
    order = jnp.argsort(flat_ids).astype(jnp.int32)
    cmp = flat_ids[None, :] < jnp.arange(1, E + 1, dtype=jnp.int32)[:, None]
    offsets_ext = jnp.concatenate(
        [jnp.zeros((1,), jnp.int32), cmp.sum(axis=1).astype(jnp.int32)]
    )
    inv_order = (
        jnp.zeros((T,), jnp.int32)
        .at[order]
        .set(jnp.arange(T, dtype=jnp.int32))
    )

    w16 = jnp.broadcast_to(
        full_topk_weight.astype(jnp.float32).reshape(T, 1), (T, 128)
    )
    xs, ws16 = _make_sc_row_and_scalar_gather(T, K, 64)(x, order, w16)
    ys = _ragged_matmul(xs, W, ws16[:, :1], offsets_ext)
    out = _make_sc_row_gather(T, H, 16)(ys, inv_order.reshape(T // 16, 16))
    return out


# speedup vs baseline: 1.0451x; 1.0044x over previous
import functools

import jax
import jax.numpy as jnp
from jax import lax
from jax.experimental import pallas as pl
from jax.experimental.pallas import tpu as pltpu
from jax.experimental.pallas import tpu_sc as plsc

_SC_CORES = 2
_SC_SUBCORES = 16
_SC_WORKERS = _SC_CORES * _SC_SUBCORES

_BLK = 32


def _make_sc_row_gather(T_rows, D, R):
    per_w = T_rows // _SC_WORKERS
    n_chunks = per_w // R
    mesh = plsc.VectorSubcoreMesh(core_axis_name="c", subcore_axis_name="s")

    @functools.partial(
        pl.kernel,
        out_type=jax.ShapeDtypeStruct((T_rows, D), jnp.float32),
        mesh=mesh,
        scratch_types=[
            pltpu.VMEM((R,), jnp.int32),
            pltpu.VMEM((R, D), jnp.float32),
            pltpu.SemaphoreType.DMA,
        ],
    )
    def gather_kernel(src_hbm, idx_hbm, out_hbm, idx_v, rows_v, sem):
        wid = lax.axis_index("s") * _SC_CORES + lax.axis_index("c")
        for c in range(n_chunks):
            base = wid * per_w + c * R
            pltpu.sync_copy(idx_hbm.at[pl.ds(base, R)], idx_v)
            pltpu.async_copy(src_hbm.at[idx_v], rows_v, sem).wait()
            pltpu.sync_copy(rows_v, out_hbm.at[pl.ds(base, R)])

    return gather_kernel


def _make_sc_row_and_scalar_gather(T_rows, D, R):
    per_w = T_rows // _SC_WORKERS
    n_chunks = per_w // R
    lanes = 128
    mesh = plsc.VectorSubcoreMesh(core_axis_name="c", subcore_axis_name="s")

    @functools.partial(
        pl.kernel,
        out_type=(
            jax.ShapeDtypeStruct((T_rows, D), jnp.float32),
            jax.ShapeDtypeStruct((T_rows, lanes), jnp.float32),
        ),
        mesh=mesh,
        scratch_types=[
            pltpu.VMEM((R,), jnp.int32),
            pltpu.VMEM((R, D), jnp.float32),
            pltpu.VMEM((R, lanes), jnp.float32),
            pltpu.SemaphoreType.DMA,
            pltpu.SemaphoreType.DMA,
        ],
    )
    def gather_kernel(src_hbm, idx_hbm, svec_hbm, rows_out, s_out,
                      idx_v, rows_v, sg_v, sem, sem2):
        wid = lax.axis_index("s") * _SC_CORES + lax.axis_index("c")
        for c in range(n_chunks):
            base = wid * per_w + c * R
            pltpu.sync_copy(idx_hbm.at[pl.ds(base, R)], idx_v)
            cp = pltpu.async_copy(src_hbm.at[idx_v], rows_v, sem)
            cp2 = pltpu.async_copy(svec_hbm.at[idx_v], sg_v, sem2)
            cp.wait()
            cp2.wait()
            pltpu.sync_copy(rows_v, rows_out.at[pl.ds(base, R)])
            pltpu.sync_copy(sg_v, s_out.at[pl.ds(base, R)])

    return gather_kernel


def _ragged_matmul_body(off_r, xs_ref, w_ref, ws_ref, o_ref):
    e = pl.program_id(0)
    start = off_r[e]
    end = off_r[e + 1]
    blk0 = start // _BLK
    n_chunks = (end + _BLK - 1) // _BLK - blk0

    def chunk(j, carry):
        s0 = (blk0 + j) * _BLK
        y = jnp.dot(
            xs_ref[pl.ds(s0, _BLK), :], w_ref[0],
            preferred_element_type=jnp.float32,
        )
        y = y * ws_ref[pl.ds(s0, _BLK), :]
        r = s0 + lax.broadcasted_iota(jnp.int32, (_BLK, 1), 0)
        mask = (r >= start) & (r < end)
        o_ref[pl.ds(s0, _BLK), :] = jnp.where(mask, y, o_ref[pl.ds(s0, _BLK), :])
        return carry

    lax.fori_loop(0, n_chunks, chunk, 0)


def _ragged_matmul(xs, W, ws, offsets_ext):
    T, K = xs.shape
    E, _, H = W.shape
    grid_spec = pltpu.PrefetchScalarGridSpec(
        num_scalar_prefetch=1,
        grid=(E,),
        in_specs=[
            pl.BlockSpec((T, K), lambda e, off: (0, 0)),
            pl.BlockSpec((1, K, H), lambda e, off: (e, 0, 0)),
            pl.BlockSpec((T, 1), lambda e, off: (0, 0)),
        ],
        out_specs=pl.BlockSpec((T, H), lambda e, off: (0, 0)),
    )
    return pl.pallas_call(
        _ragged_matmul_body,
        grid_spec=grid_spec,
        out_shape=jax.ShapeDtypeStruct((T, H), jnp.float32),
    )(offsets_ext, xs, W, ws)


def kernel(intermediate_states, down_weight, full_topk_ids, full_topk_weight):
    x = intermediate_states
    W = down_weight
    T, K = x.shape
    E, _, H = W.shape

    flat_ids = full_topk_ids.reshape(T).astype(jnp.int32)
    order = jnp.argsort(flat_ids).astype(jnp.int32)
    cmp = flat_ids[None, :] < jnp.arange(1, E + 1, dtype=jnp.int32)[:, None]
    offsets_ext = jnp.concatenate(
        [jnp.zeros((1,), jnp.int32), cmp.sum(axis=1).astype(jnp.int32)]
    )
    inv_order = (
        jnp.zeros((T,), jnp.int32)
        .at[order]
        .set(jnp.arange(T, dtype=jnp.int32))
    )

    w16 = jnp.broadcast_to(
        full_topk_weight.astype(jnp.float32).reshape(T, 1), (T, 128)
    )
    xs, ws16 = _make_sc_row_and_scalar_gather(T, K, 64)(x, order, w16)
    ys = _ragged_matmul(xs, W, ws16[:, :1], offsets_ext)
    out = _make_sc_row_gather(T, H, 32)(ys, inv_order)
    return out
